# baseline (speedup 1.0000x reference)
import numpy as np

import jax
import jax.numpy as jnp
from jax.experimental import pallas as pl
from jax.experimental.pallas import tpu as pltpu

_D = 8

_SRCS = [(2, -1), (3, -1), (0, 0), (1, 0), (2, 0), (3, 0), (0, 1), (1, 1)]


def _make_kernel(B):
    R3 = B * 7

    RH = R3 // 2

    def body(xb_ref, w1_ref, w2a_ref, w2b_ref, b2a_ref, b2b_ref,
             wf_ref, blt_ref, feat_ref, logit_ref, xq, fsp):
        f32 = jnp.float32
        bf16 = jnp.bfloat16
        h7 = jax.lax.broadcasted_iota(jnp.int32, (RH, 1), 0) % 7

        def shifted(src, r0, e):
            s = src[r0 + _D + e:r0 + _D + e + RH, :]
            if e < 0:
                s = jnp.where(h7 >= -e, s, jnp.bfloat16(0))
            elif e > 0:
                s = jnp.where(h7 <= 6 - e, s, jnp.bfloat16(0))
            return s

        xq[0:_D, :] = jnp.zeros((_D, 448), bf16)
        xq[_D + RH:_D + RH + 16, :] = jnp.zeros((16, 448), bf16)
        xq[R3 + 24:, :] = jnp.zeros((8, 448), bf16)
        fsp[RH:RH + 8, :] = jnp.zeros((8, 224), bf16)
        fsp[R3 + 8:, :] = jnp.zeros((8, 224), bf16)

        for half in range(2):
            r0 = half * RH
            q0 = half * (RH + 16)
            f0 = half * (RH + 8)

            acc1 = jnp.dot(xb_ref[r0:r0 + RH, :], w1_ref[...],
                           preferred_element_type=f32)
            ybf = jnp.maximum(acc1, 0.0).astype(bf16)

            fa = jnp.maximum(ybf[:, 0:448], ybf[:, 896:1344])
            fb = jnp.maximum(ybf[:, 448:896], ybf[:, 1344:1792])
            xq[q0 + _D:q0 + _D + RH, :] = jnp.maximum(fa, fb)

            acc2a = acc2b = None
            for ei, e in enumerate((-1, 0, 1)):
                s = shifted(xq, q0, e)
                pa = jnp.dot(s[:, 0:384], w2a_ref[ei],
                             preferred_element_type=f32)
                pb = jnp.dot(s[:, 192:448], w2b_ref[ei],
                             preferred_element_type=f32)
                acc2a = pa if acc2a is None else acc2a + pa
                acc2b = pb if acc2b is None else acc2b + pb
            y2a = jnp.maximum(acc2a + b2a_ref[...], 0.0)
            y2b = jnp.maximum(acc2b + b2b_ref[...], 0.0)

            fha = jnp.maximum(jnp.maximum(y2a[:, 0:128], y2a[:, 256:384]),
                              jnp.maximum(y2a[:, 128:256], y2a[:, 384:512]))
            fhb = jnp.maximum(jnp.maximum(y2b[:, 0:128], y2b[:, 256:384]),
                              jnp.maximum(y2b[:, 128:256], y2b[:, 384:512]))
            feat_ref[r0:r0 + RH, 0:128] = fha
            feat_ref[r0:r0 + RH, 128:224] = fhb[:, 0:96]
            fsp[f0:f0 + RH, 0:128] = fha.astype(bf16)
            fsp[f0:f0 + RH, 128:224] = fhb[:, 0:96].astype(bf16)

            pf = jnp.dot(fsp[f0:f0 + RH + 8, :], wf_ref[...],
                         preferred_element_type=f32)
            acc = pf[0:RH, 0:128]
            for h in range(1, 7):
                acc = acc + pf[h:h + RH, 128 * h:128 * h + 128]
            logit_ref[r0:r0 + RH, :] = acc + blt_ref[...]

    return body, R3


def _forward(xb, w1, w2a, w2b, b2a, b2b, wf, blt):
    n = xb.shape[0] // 7
    B = 256 if n % 256 == 0 else (64 if n % 64 == 0 else n)
    body, R3 = _make_kernel(B)
    bf16 = jnp.bfloat16

    feat_k, logit_k = pl.pallas_call(
        body,
        out_shape=(jax.ShapeDtypeStruct((n * 7, 224), jnp.float32),
                   jax.ShapeDtypeStruct((n * 7, 128), jnp.float32)),
        grid=(n // B,),
        in_specs=[
            pl.BlockSpec((R3, 256), lambda i: (i, 0)),
            pl.BlockSpec((256, 1792), lambda i: (0, 0)),
            pl.BlockSpec((3, 384, 512), lambda i: (0, 0, 0)),
            pl.BlockSpec((3, 256, 512), lambda i: (0, 0, 0)),
            pl.BlockSpec((1, 512), lambda i: (0, 0)),
            pl.BlockSpec((1, 512), lambda i: (0, 0)),
            pl.BlockSpec((224, 896), lambda i: (0, 0)),
            pl.BlockSpec((1, 128), lambda i: (0, 0)),
        ],
        out_specs=(
            pl.BlockSpec((R3, 224), lambda i: (i, 0)),
            pl.BlockSpec((R3, 128), lambda i: (i, 0)),
        ),
        scratch_shapes=(
            [pltpu.VMEM((R3 + 32, 448), bf16),
             pltpu.VMEM((R3 + 16, 224), bf16)]
        ),
        compiler_params=pltpu.CompilerParams(
            dimension_semantics=("parallel",),
            vmem_limit_bytes=56 * 1024 * 1024),
    )(xb, w1, w2a, w2b, b2a, b2b, wf, blt)
    return feat_k, logit_k


@jax.jit
def kernel(x, a1, b1t, a2, b2t, wlp, blt):
    n = x.shape[0]
    x2d = x.reshape(n, 28, 28).astype(jnp.bfloat16)
    xs = [x2d[:, m::4, :] for m in range(4)]

    zrow = jnp.zeros((n, 1, 28), jnp.bfloat16)
    pieces = []
    for c, e in _SRCS:
        if e == -1:
            p = jnp.concatenate([zrow, xs[c][:, :6, :]], 1)
        elif e == 1:
            p = jnp.concatenate([xs[c][:, 1:, :], zrow], 1)
        else:
            p = xs[c]
        pieces.append(jnp.pad(p, ((0, 0), (0, 0), (0, 4))))
    xb = jnp.concatenate(pieces, 2).reshape(n * 7, 256)
    xb = xb.at[:, 255].set(jnp.bfloat16(1))

    a1blk = jnp.pad(a1[:, 2:30, :], ((0, 0), (0, 4), (0, 0)))
    w1o = jnp.zeros((8, 32, 4, 448), jnp.float32)
    for m in range(4):
        for i in range(5):
            s = _SRCS.index(((m + i - 2) % 4, (m + i - 2) // 4))
            w1o = w1o.at[s, :, m, :].set(a1blk[i])
    w1o = w1o.reshape(256, 4 * 448)
    w1o = w1o.at[255, :].set(jnp.tile(b1t, (1, 4)).reshape(4 * 448))
    idx1 = np.zeros(1792, np.int64)
    for p in range(4):
        for jj in range(448):
            q, ci = jj // 32, jj % 32
            m = (0, 2)[ci >= 16] if p < 2 else (1, 3)[ci >= 16]
            l = 32 * q + ci % 16 + (16 if p % 2 == 1 else 0)
            idx1[448 * p + jj] = m * 448 + l
    w1 = w1o[:, idx1].astype(jnp.bfloat16)

    t = a2[:, 32:256, :].reshape(5, 14, 16, 448)
    zb = jnp.zeros((14, 16, 448), jnp.float32)
    idxh = np.zeros(512, np.int64)
    valh = np.zeros(512, np.float32)
    for p in range(2):
        for jj in range(224):
            qp, co = jj // 32, jj % 32
            idxh[256 * p + jj] = 64 * qp + co + 32 * p
            valh[256 * p + jj] = 1.0
    w2es = []
    for e in (-1, 0, 1):
        halves = []
        for v in range(2):
            ie, io = 2 * e + 2 - v, 2 * e + 3 - v
            even = t[ie] if 0 <= ie <= 4 else zb
            odd = t[io] if 0 <= io <= 4 else zb
            blk = jnp.concatenate([even, odd], 1).reshape(448, 448)
            halves.append(blk[:, idxh] * valh)
        w2es.append(jnp.pad(jnp.concatenate(halves, 1), ((0, 64), (0, 0))))
    w2 = jnp.stack(w2es)
    w2blk = w2.reshape(3, 512, 4, 256)
    w2a = w2blk[:, 0:384, :, 0:128].reshape(3, 384, 512)
    w2b = w2blk[:, 192:448, :, 128:256].reshape(3, 256, 512)
    w2a = w2a.astype(jnp.bfloat16)
    w2b = w2b.astype(jnp.bfloat16)
    b2h = b2t.reshape(448)[idxh] * valh
    b2 = jnp.concatenate([b2h, b2h]).reshape(1, 1024)
    b2blk = b2.reshape(1, 4, 256)
    b2a = b2blk[:, :, 0:128].reshape(1, 512)
    b2b = b2blk[:, :, 128:256].reshape(1, 512)

    wf = jnp.transpose(wlp, (1, 0, 2)).reshape(224, 896)
    wf = wf.astype(jnp.bfloat16)
    feat_k, logit_k = _forward(xb, w1, w2a, w2b, b2a, b2b, wf, blt)
    feat = feat_k.reshape(n, 7, 7, 32).transpose(0, 3, 1, 2).reshape(n, 1568)
    logits = logit_k[0::7, :10]
    return logits, feat


# speedup vs baseline: 1.1124x; 1.1124x over previous
import numpy as np

import jax
import jax.numpy as jnp
from jax.experimental import pallas as pl
from jax.experimental.pallas import tpu as pltpu

_D = 8

_SRCS = [(2, -1), (3, -1), (0, 0), (1, 0), (2, 0), (3, 0), (0, 1), (1, 1)]


def _make_kernel(B):
    R3 = B * 7

    RH = R3 // 2

    def body(xb_ref, w1_ref, b1_ref, w2a_ref, w2b_ref, b2a_ref, b2b_ref,
             wf_ref, blt_ref, feat_ref, logit_ref, xq, fsp):
        f32 = jnp.float32
        bf16 = jnp.bfloat16
        h7 = jax.lax.broadcasted_iota(jnp.int32, (RH, 1), 0) % 7

        def shifted(src, r0, e):
            s = src[r0 + _D + e:r0 + _D + e + RH, :]
            if e < 0:
                s = jnp.where(h7 >= -e, s, jnp.bfloat16(0))
            elif e > 0:
                s = jnp.where(h7 <= 6 - e, s, jnp.bfloat16(0))
            return s

        xq[0:_D, :] = jnp.zeros((_D, 448), bf16)
        xq[_D + RH:_D + RH + 16, :] = jnp.zeros((16, 448), bf16)
        xq[R3 + 24:, :] = jnp.zeros((8, 448), bf16)
        fsp[RH:RH + 8, :] = jnp.zeros((8, 224), bf16)
        fsp[R3 + 8:, :] = jnp.zeros((8, 224), bf16)

        for half in range(2):
            r0 = half * RH
            q0 = half * (RH + 16)
            f0 = half * (RH + 8)

            acc1 = jnp.dot(xb_ref[r0:r0 + RH, :], w1_ref[...],
                           preferred_element_type=f32)
            ybf = jnp.maximum(acc1 + b1_ref[...], 0.0).astype(bf16)

            fa = jnp.maximum(ybf[:, 0:448], ybf[:, 896:1344])
            fb = jnp.maximum(ybf[:, 448:896], ybf[:, 1344:1792])
            xq[q0 + _D:q0 + _D + RH, :] = jnp.maximum(fa, fb)

            acc2a = acc2b = None
            for ei, e in enumerate((-1, 0, 1)):
                s = shifted(xq, q0, e)
                pa = jnp.dot(s[:, 0:384], w2a_ref[ei],
                             preferred_element_type=f32)
                pb = jnp.dot(s[:, 192:448], w2b_ref[ei],
                             preferred_element_type=f32)
                acc2a = pa if acc2a is None else acc2a + pa
                acc2b = pb if acc2b is None else acc2b + pb
            y2a = jnp.maximum(acc2a + b2a_ref[...], 0.0)
            y2b = jnp.maximum(acc2b + b2b_ref[...], 0.0)

            fha = jnp.maximum(jnp.maximum(y2a[:, 0:128], y2a[:, 256:384]),
                              jnp.maximum(y2a[:, 128:256], y2a[:, 384:512]))
            fhb = jnp.maximum(jnp.maximum(y2b[:, 0:128], y2b[:, 256:384]),
                              jnp.maximum(y2b[:, 128:256], y2b[:, 384:512]))
            feat_ref[r0:r0 + RH, 0:128] = fha
            feat_ref[r0:r0 + RH, 128:224] = fhb[:, 0:96]
            fsp[f0:f0 + RH, 0:128] = fha.astype(bf16)
            fsp[f0:f0 + RH, 128:224] = fhb[:, 0:96].astype(bf16)

            pf = jnp.dot(fsp[f0:f0 + RH + 8, :], wf_ref[...],
                         preferred_element_type=f32)
            acc = pf[0:RH, 0:128]
            for h in range(1, 7):
                acc = acc + pf[h:h + RH, 128 * h:128 * h + 128]
            logit_ref[r0:r0 + RH, :] = acc + blt_ref[...]

    return body, R3


def _forward(xb, w1, b1, w2a, w2b, b2a, b2b, wf, blt):
    n = xb.shape[0] // 7
    B = 256 if n % 256 == 0 else (64 if n % 64 == 0 else n)
    body, R3 = _make_kernel(B)
    bf16 = jnp.bfloat16

    feat_k, logit_k = pl.pallas_call(
        body,
        out_shape=(jax.ShapeDtypeStruct((n * 7, 224), jnp.float32),
                   jax.ShapeDtypeStruct((n * 7, 128), jnp.float32)),
        grid=(n // B,),
        in_specs=[
            pl.BlockSpec((R3, 256), lambda i: (i, 0)),
            pl.BlockSpec((256, 1792), lambda i: (0, 0)),
            pl.BlockSpec((1, 1792), lambda i: (0, 0)),
            pl.BlockSpec((3, 384, 512), lambda i: (0, 0, 0)),
            pl.BlockSpec((3, 256, 512), lambda i: (0, 0, 0)),
            pl.BlockSpec((1, 512), lambda i: (0, 0)),
            pl.BlockSpec((1, 512), lambda i: (0, 0)),
            pl.BlockSpec((224, 896), lambda i: (0, 0)),
            pl.BlockSpec((1, 128), lambda i: (0, 0)),
        ],
        out_specs=(
            pl.BlockSpec((R3, 224), lambda i: (i, 0)),
            pl.BlockSpec((R3, 128), lambda i: (i, 0)),
        ),
        scratch_shapes=(
            [pltpu.VMEM((R3 + 32, 448), bf16),
             pltpu.VMEM((R3 + 16, 224), bf16)]
        ),
        compiler_params=pltpu.CompilerParams(
            dimension_semantics=("parallel",),
            vmem_limit_bytes=56 * 1024 * 1024),
    )(xb, w1, b1, w2a, w2b, b2a, b2b, wf, blt)
    return feat_k, logit_k


@jax.jit
def kernel(x, a1, b1t, a2, b2t, wlp, blt):
    n = x.shape[0]
    x2d = x.reshape(n, 28, 28).astype(jnp.bfloat16)
    xs = [x2d[:, m::4, :] for m in range(4)]

    zrow = jnp.zeros((n, 1, 28), jnp.bfloat16)
    pieces = []
    for c, e in _SRCS:
        if e == -1:
            p = jnp.concatenate([zrow, xs[c][:, :6, :]], 1)
        elif e == 1:
            p = jnp.concatenate([xs[c][:, 1:, :], zrow], 1)
        else:
            p = xs[c]
        pieces.append(jnp.pad(p, ((0, 0), (0, 0), (0, 4))))
    xb = jnp.concatenate(pieces, 2).reshape(n * 7, 256)

    a1blk = jnp.pad(a1[:, 2:30, :], ((0, 0), (0, 4), (0, 0)))
    w1o = jnp.zeros((8, 32, 4, 448), jnp.float32)
    for m in range(4):
        for i in range(5):
            s = _SRCS.index(((m + i - 2) % 4, (m + i - 2) // 4))
            w1o = w1o.at[s, :, m, :].set(a1blk[i])
    w1o = w1o.reshape(256, 4 * 448)
    idx1 = np.zeros(1792, np.int64)
    for p in range(4):
        for jj in range(448):
            q, ci = jj // 32, jj % 32
            m = (0, 2)[ci >= 16] if p < 2 else (1, 3)[ci >= 16]
            l = 32 * q + ci % 16 + (16 if p % 2 == 1 else 0)
            idx1[448 * p + jj] = m * 448 + l
    w1 = w1o[:, idx1].astype(jnp.bfloat16)
    b1f = jnp.tile(b1t, (1, 4)).reshape(4 * 448)
    b1 = b1f[idx1].reshape(1, 1792)

    t = a2[:, 32:256, :].reshape(5, 14, 16, 448)
    zb = jnp.zeros((14, 16, 448), jnp.float32)
    idxh = np.zeros(512, np.int64)
    valh = np.zeros(512, np.float32)
    for p in range(2):
        for jj in range(224):
            qp, co = jj // 32, jj % 32
            idxh[256 * p + jj] = 64 * qp + co + 32 * p
            valh[256 * p + jj] = 1.0
    w2es = []
    for e in (-1, 0, 1):
        halves = []
        for v in range(2):
            ie, io = 2 * e + 2 - v, 2 * e + 3 - v
            even = t[ie] if 0 <= ie <= 4 else zb
            odd = t[io] if 0 <= io <= 4 else zb
            blk = jnp.concatenate([even, odd], 1).reshape(448, 448)
            halves.append(blk[:, idxh] * valh)
        w2es.append(jnp.pad(jnp.concatenate(halves, 1), ((0, 64), (0, 0))))
    w2 = jnp.stack(w2es)
    w2blk = w2.reshape(3, 512, 4, 256)
    w2a = w2blk[:, 0:384, :, 0:128].reshape(3, 384, 512)
    w2b = w2blk[:, 192:448, :, 128:256].reshape(3, 256, 512)
    w2a = w2a.astype(jnp.bfloat16)
    w2b = w2b.astype(jnp.bfloat16)
    b2h = b2t.reshape(448)[idxh] * valh
    b2 = jnp.concatenate([b2h, b2h]).reshape(1, 1024)
    b2blk = b2.reshape(1, 4, 256)
    b2a = b2blk[:, :, 0:128].reshape(1, 512)
    b2b = b2blk[:, :, 128:256].reshape(1, 512)

    wf = jnp.transpose(wlp, (1, 0, 2)).reshape(224, 896)
    wf = wf.astype(jnp.bfloat16)
    feat_k, logit_k = _forward(xb, w1, b1, w2a, w2b, b2a, b2b, wf, blt)
    feat = feat_k.reshape(n, 7, 7, 32).transpose(0, 3, 1, 2).reshape(n, 1568)
    logits = logit_k[0::7, :10]
    return logits, feat
